# Initial kernel scaffold; baseline (speedup 1.0000x reference)
import jax
import jax.numpy as jnp
from jax.experimental import pallas as pl


def kernel(x, w1, b1, s1, t1, w2, b2, s2, t2, w3, b3, s3, t3, w4, b4, s4, t4, w5, b5, s5, t5, fc1_w, fc1_b, fc2_w, fc2_b):
    raise NotImplementedError("write your pallas kernel here")


# speedup vs baseline: 16.5623x; 16.5623x over previous
import jax
import jax.numpy as jnp
from jax.experimental import pallas as pl
from jax.experimental.pallas import tpu as pltpu


def _pool_affine(r, Wp, W, s, t):
    M, Cout = r.shape
    Qp, Q = Wp // 2, W // 2
    rb = r.astype(jnp.bfloat16)
    rx = jnp.concatenate([rb, jnp.zeros((8, Cout), jnp.bfloat16)], axis=0)
    ms = jnp.maximum(rx[0:M], rx[1:M + 1])
    v = pltpu.bitcast(ms, jnp.int32)
    mc = pltpu.bitcast(v << 16, jnp.float32)
    h2 = M // Wp // 2
    m4 = mc.reshape(h2, 2, Qp, Cout)
    m = jnp.maximum(m4[:, 0], m4[:, 1])[:, 0:Q]
    return (m * s + t).astype(jnp.bfloat16)


def _conv_stage(A, wc_ref, b, s, t, tiles):
    H, W, C = A.shape
    Cout = wc_ref.shape[-1]
    Wp = ((W + 2 + 15) // 16) * 16
    z_top = jnp.zeros((1, W, C), jnp.bfloat16)
    z_bot = jnp.zeros((2, W, C), jnp.bfloat16)
    xp = jnp.concatenate([z_top, A, z_bot], axis=0)
    z_col_l = jnp.zeros((H + 3, 1, C), jnp.bfloat16)
    z_col_r = jnp.zeros((H + 3, Wp - W - 1, C), jnp.bfloat16)
    xp = jnp.concatenate([z_col_l, xp, z_col_r], axis=1)
    F = xp.reshape((H + 3) * Wp, C)
    L = (H + 2) * Wp
    G = jnp.concatenate([F[0:L], F[1:L + 1], F[2:L + 2]], axis=1)
    w0, w1, w2 = wc_ref[0], wc_ref[1], wc_ref[2]

    P = H // 2
    Ppt = P // tiles
    pieces = []
    for ti in range(tiles):
        base = 2 * ti * Ppt * Wp
        M = 2 * Ppt * Wp
        acc = jnp.dot(G[base:base + M], w0,
                      preferred_element_type=jnp.float32)
        acc += jnp.dot(G[base + Wp:base + Wp + M], w1,
                       preferred_element_type=jnp.float32)
        acc += jnp.dot(G[base + 2 * Wp:base + 2 * Wp + M], w2,
                       preferred_element_type=jnp.float32)
        r = jnp.maximum(acc + b, 0.0)
        pieces.append(_pool_affine(r, Wp, W, s, t))
    return jnp.concatenate(pieces, axis=0) if tiles > 1 else pieces[0]


def _net_kernel(g1_ref,
                wc1_ref, b1_ref, s1_ref, t1_ref,
                wc2_ref, b2_ref, s2_ref, t2_ref,
                wc3_ref, b3_ref, s3_ref, t3_ref,
                wc4_ref, b4_ref, s4_ref, t4_ref,
                wc5_ref, b5_ref, s5_ref, t5_ref,
                fc1w_ref, fc1b_ref, fc2w_ref, fc2b_ref,
                o_ref):
    F1 = g1_ref[0].reshape(226 * 224, 9)
    w10, w11, w12 = wc1_ref[0], wc1_ref[1], wc1_ref[2]
    b1, s1, t1 = b1_ref[...], s1_ref[...], t1_ref[...]
    pieces = []
    for ti in range(4):
        base = ti * 56 * 224
        M = 56 * 224
        acc = jnp.dot(F1[base:base + M], w10,
                      preferred_element_type=jnp.float32)
        acc += jnp.dot(F1[base + 224:base + 224 + M], w11,
                       preferred_element_type=jnp.float32)
        acc += jnp.dot(F1[base + 448:base + 448 + M], w12,
                       preferred_element_type=jnp.float32)
        r = jnp.maximum(acc + b1, 0.0)
        pieces.append(_pool_affine(r, 224, 224, s1, t1))
    A = jnp.concatenate(pieces, axis=0)

    A = _conv_stage(A, wc2_ref, b2_ref[...], s2_ref[...], t2_ref[...], 2)
    A = _conv_stage(A, wc3_ref, b3_ref[...], s3_ref[...], t3_ref[...], 1)
    A = _conv_stage(A, wc4_ref, b4_ref[...], s4_ref[...], t4_ref[...], 1)
    A = _conv_stage(A, wc5_ref, b5_ref[...], s5_ref[...], t5_ref[...], 1)

    flat = A.reshape(1, 7 * 7 * 512)
    y1 = jnp.dot(flat, fc1w_ref[...], preferred_element_type=jnp.float32)
    y1 = jnp.maximum(y1 + fc1b_ref[...], 0.0)
    y2 = jnp.dot(y1.astype(jnp.bfloat16), fc2w_ref[...],
                 preferred_element_type=jnp.float32)
    o_ref[0] = y2 + fc2b_ref[...]


def kernel(x, w1, b1, s1, t1, w2, b2, s2, t2, w3, b3, s3, t3,
           w4, b4, s4, t4, w5, b5, s5, t5, fc1_w, fc1_b, fc2_w, fc2_b):
    N = x.shape[0]
    xt = jnp.transpose(x, (0, 2, 3, 1)).astype(jnp.bfloat16)
    xp = jnp.pad(xt, ((0, 0), (1, 1), (1, 1), (0, 0)))
    g1 = jnp.concatenate([xp[:, :, d:d + 224, :] for d in range(3)],
                         axis=3)

    wcs = [w.reshape(3, 3 * cin, cout) for w, cin, cout in
           ((w1, 3, 32), (w2, 32, 64), (w3, 64, 128),
            (w4, 128, 256), (w5, 256, 512))]
    f32 = jnp.float32
    row = lambda v: v.reshape(1, -1).astype(f32)
    stage_params = []
    for wc, b, s, t in zip(wcs, (b1, b2, b3, b4, b5), (s1, s2, s3, s4, s5),
                           (t1, t2, t3, t4, t5)):
        stage_params += [wc, row(b), row(s), row(t)]

    full = lambda shape: pl.BlockSpec(shape, lambda n: (0,) * len(shape))
    in_specs = [pl.BlockSpec((1, 226, 224, 9), lambda n: (n, 0, 0, 0))]
    for wc, b, s, t in zip(stage_params[0::4], stage_params[1::4],
                           stage_params[2::4], stage_params[3::4]):
        in_specs += [full(wc.shape), full(b.shape), full(s.shape),
                     full(t.shape)]
    in_specs += [full(fc1_w.shape), full((1, 128)), full(fc2_w.shape),
                 full((1, 1000))]

    out = pl.pallas_call(
        _net_kernel,
        out_shape=jax.ShapeDtypeStruct((N, 1, 1000), jnp.float32),
        grid=(N,),
        in_specs=in_specs,
        out_specs=pl.BlockSpec((1, 1, 1000), lambda n: (n, 0, 0)),
        compiler_params=pltpu.CompilerParams(
            dimension_semantics=("parallel",)),
    )(g1, *stage_params, fc1_w, row(fc1_b), fc2_w, row(fc2_b))
    return out.reshape(N, 1000)
